# Initial kernel scaffold; baseline (speedup 1.0000x reference)
import jax
import jax.numpy as jnp
from jax.experimental import pallas as pl


def kernel(hidden_states, Wg, bias, W1, W3, W2, Ws1, Ws3, Ws2):
    raise NotImplementedError("write your pallas kernel here")


# speedup vs baseline: 1.5528x; 1.5528x over previous
import functools

import jax
import jax.numpy as jnp
from jax import lax
from jax.experimental import pallas as pl
from jax.experimental.pallas import tpu as pltpu
from jax.experimental.pallas import tpu_sc as plsc

T = 2048
H = 1024
E = 16
I = 256
TOPK = 2
NG = 4
TG = 2
IS = 256
RSF = 2.5

NC = 2
NS = 16
NW = NC * NS
L = 16
TPW = T // NW
GPW = TPW // L
BLK = 128
HBLK = T * TOPK // (NC * BLK) + E
NBLK = NC * HBLK
HR = HBLK * BLK
NROWS = NBLK * BLK
NEG = -1e30
TA = 256


def _a_body(x_ref, wg_ref, b_ref, ws1_ref, ws3_ref, ws2_ref,
            sco_ref, sfc_ref, shared_ref):
    dn = (((1,), (1,)), ((), ()))
    x = x_ref[...]
    h1 = lax.dot_general(x, ws1_ref[...], dn, preferred_element_type=jnp.float32)
    h3 = lax.dot_general(x, ws3_ref[...], dn, preferred_element_type=jnp.float32)
    h = h1 * jax.nn.sigmoid(h1) * h3
    shared_ref[...] = lax.dot_general(h, ws2_ref[...], dn,
                                      preferred_element_type=jnp.float32)
    logits = lax.dot_general(x, wg_ref[...], dn, preferred_element_type=jnp.float32)
    s = jax.nn.sigmoid(logits)
    sco_ref[...] = s
    sfc_ref[...] = s + b_ref[...]


def _run_a(x, wg, bias2d, ws1, ws3, ws2):
    return pl.pallas_call(
        _a_body,
        grid=(T // TA,),
        in_specs=[
            pl.BlockSpec((TA, H), lambda i: (i, 0)),
            pl.BlockSpec((E, H), lambda i: (0, 0)),
            pl.BlockSpec((1, E), lambda i: (0, 0)),
            pl.BlockSpec((IS, H), lambda i: (0, 0)),
            pl.BlockSpec((IS, H), lambda i: (0, 0)),
            pl.BlockSpec((H, IS), lambda i: (0, 0)),
        ],
        out_specs=[
            pl.BlockSpec((TA, E), lambda i: (i, 0)),
            pl.BlockSpec((TA, E), lambda i: (i, 0)),
            pl.BlockSpec((TA, H), lambda i: (i, 0)),
        ],
        out_shape=[
            jax.ShapeDtypeStruct((T, E), jnp.float32),
            jax.ShapeDtypeStruct((T, E), jnp.float32),
            jax.ShapeDtypeStruct((T, H), jnp.float32),
        ],
    )(x, wg, bias2d, ws1, ws3, ws2)


def _b_body(sfc_hbm, sco_hbm, x_hbm,
            xs_hbm, wts_hbm, rowt_hbm, meta_hbm, cntx_hbm,
            sfc_v, sco_v, xrows_v, idx1_v, idx2_v, wa_v, wb_v,
            p1_v, p2_v, e1_v, e2_v, cnt_v, all_v,
            mbe_v, mac_v, sem):
    core = lax.axis_index("c")
    s = lax.axis_index("s")
    wid = core * NS + s
    base = wid * TPW
    pltpu.sync_copy(sfc_hbm.at[pl.ds(base * L, TPW * L)], sfc_v)
    pltpu.sync_copy(sco_hbm.at[pl.ds(base * L, TPW * L)], sco_v)
    pltpu.sync_copy(x_hbm.at[pl.ds(base, TPW)], xrows_v)

    iota = lax.iota(jnp.int32, L)
    zero = jnp.zeros((L,), jnp.int32)
    cnts = [jnp.int32(0)] * E

    for g in range(GPW):
        col0 = (g * L) * L
        S = []
        SR = []
        for e in range(E):
            idx = iota * L + (col0 + e)
            S.append(plsc.load_gather(sfc_v, [idx]))
            SR.append(plsc.load_gather(sco_v, [idx]))
        G = []
        for gg in range(NG):
            v = S[4 * gg:4 * gg + 4]
            m = None
            for a in range(4):
                for b in range(a + 1, 4):
                    s_ = v[a] + v[b]
                    m = s_ if m is None else jnp.maximum(m, s_)
            G.append(m)
        gsel = []
        for gg in range(NG):
            rank = None
            for gp in range(NG):
                if gp == gg:
                    continue
                c = (G[gp] >= G[gg]) if gp < gg else (G[gp] > G[gg])
                ci = c.astype(jnp.int32)
                rank = ci if rank is None else rank + ci
            gsel.append(rank < TG)
        M = [jnp.where(gsel[e // 4], S[e], NEG) for e in range(E)]
        M1 = M[0]
        for m_ in M[1:]:
            M1 = jnp.maximum(M1, m_)
        i1 = jnp.full((L,), E, jnp.int32)
        for e in range(E - 1, -1, -1):
            i1 = jnp.where(M[e] == M1, e, i1)
        w1 = jnp.zeros((L,), jnp.float32)
        for e in range(E):
            w1 = jnp.where(i1 == e, SR[e], w1)
        M2l = [jnp.where(i1 == e, NEG, M[e]) for e in range(E)]
        M2 = M2l[0]
        for m_ in M2l[1:]:
            M2 = jnp.maximum(M2, m_)
        i2 = jnp.full((L,), E, jnp.int32)
        for e in range(E - 1, -1, -1):
            i2 = jnp.where(M2l[e] == M2, e, i2)
        w2 = jnp.zeros((L,), jnp.float32)
        for e in range(E):
            w2 = jnp.where(i2 == e, SR[e], w2)
        ws = w1 + w2 + 1e-20
        w1n = w1 * RSF / ws
        w2n = w2 * RSF / ws
        wa_v[pl.ds(g * L, L)] = w1n
        wb_v[pl.ds(g * L, L)] = w2n
        pos1 = jnp.zeros((L,), jnp.int32)
        pos2 = jnp.zeros((L,), jnp.int32)
        for e in range(E):
            m1 = i1 == e
            m2 = i2 == e
            occ = (m1 | m2).astype(jnp.int32)
            cums = plsc.cumsum(occ)
            pe = cnts[e] + cums - 1
            pos1 = jnp.where(m1, pe, pos1)
            pos2 = jnp.where(m2, pe, pos2)
            cnts[e] = cnts[e] + jnp.sum(occ)
        p1_v[pl.ds(g * L, L)] = pos1
        p2_v[pl.ds(g * L, L)] = pos2
        e1_v[pl.ds(g * L, L)] = i1
        e2_v[pl.ds(g * L, L)] = i2

    cv = jnp.zeros((L,), jnp.int32)
    for e in range(E):
        cv = jnp.where(iota == e, cnts[e], cv)
    cnt_v[...] = cv
    pltpu.sync_copy(cnt_v, cntx_hbm.at[wid])
    plsc.subcore_barrier()
    pltpu.sync_copy(cntx_hbm.at[pl.ds(core * NS, NS)], all_v)
    tb = jnp.zeros((L,), jnp.int32)
    tot = jnp.zeros((L,), jnp.int32)
    sv = jnp.full((L,), s, jnp.int32)
    for w in range(NS):
        rw = all_v[w, :]
        tot = tot + rw
        tb = tb + jnp.where(sv > w, rw, zero)
    szb = ((tot + (BLK - 1)) // BLK) * BLK
    co = plsc.cumsum(szb)
    basev = co - szb + tb + core * HR

    gdn = lax.GatherDimensionNumbers(offset_dims=(), collapsed_slice_dims=(0,),
                                     start_index_map=(0,))

    def vgather(vec, idx):
        return lax.gather(vec, idx.reshape(L, 1), gdn, (1,),
                          mode=lax.GatherScatterMode.PROMISE_IN_BOUNDS)

    for g in range(GPW):
        i1 = e1_v[pl.ds(g * L, L)]
        i2 = e2_v[pl.ds(g * L, L)]
        r1 = p1_v[pl.ds(g * L, L)] + vgather(basev, i1)
        r2 = p2_v[pl.ds(g * L, L)] + vgather(basev, i2)
        idx1_v[pl.ds(g * L, L)] = jnp.clip(r1, 0, NROWS - 1)
        idx2_v[pl.ds(g * L, L)] = jnp.clip(r2, 0, NROWS - 1)

    cp1 = pltpu.async_copy(xrows_v, xs_hbm.at[idx1_v], sem)
    cp1.wait()
    cp2 = pltpu.async_copy(xrows_v, xs_hbm.at[idx2_v], sem)
    cp2.wait()
    pltpu.sync_copy(idx1_v, rowt_hbm.at[0, pl.ds(base, TPW)])
    pltpu.sync_copy(idx2_v, rowt_hbm.at[1, pl.ds(base, TPW)])
    pltpu.sync_copy(wa_v, wts_hbm.at[0, pl.ds(base, TPW)])
    pltpu.sync_copy(wb_v, wts_hbm.at[1, pl.ds(base, TPW)])

    omax = jnp.max(co)
    for vb in range(HBLK // L):
        brow = (vb * L + iota) * BLK
        be = jnp.zeros((L,), jnp.int32)
        for e in range(E):
            oe = vgather(co, jnp.full((L,), e, jnp.int32))
            be = be + (oe <= brow).astype(jnp.int32)
        be = jnp.minimum(be, E - 1)
        mbe_v[pl.ds(vb * L, L)] = be
        mac_v[pl.ds(vb * L, L)] = (brow < omax).astype(jnp.int32)

    @pl.when(s == 0)
    def _():
        pltpu.sync_copy(mbe_v, meta_hbm.at[pl.ds(core * HBLK, HBLK)])
        pltpu.sync_copy(mac_v, meta_hbm.at[pl.ds(NBLK + core * HBLK, HBLK)])


def _run_b(sfc_flat, sco_flat, x):
    mesh = plsc.VectorSubcoreMesh(core_axis_name="c", subcore_axis_name="s")
    f = pl.kernel(
        _b_body,
        out_type=[
            jax.ShapeDtypeStruct((NROWS, H), jnp.float32),
            jax.ShapeDtypeStruct((2, T), jnp.float32),
            jax.ShapeDtypeStruct((2, T), jnp.int32),
            jax.ShapeDtypeStruct((2 * NBLK,), jnp.int32),
            jax.ShapeDtypeStruct((NW, L), jnp.int32),
        ],
        mesh=mesh,
        compiler_params=pltpu.CompilerParams(needs_layout_passes=False),
        scratch_types=[
            pltpu.VMEM((TPW * L,), jnp.float32),
            pltpu.VMEM((TPW * L,), jnp.float32),
            pltpu.VMEM((TPW, H), jnp.float32),
            pltpu.VMEM((TPW,), jnp.int32),
            pltpu.VMEM((TPW,), jnp.int32),
            pltpu.VMEM((TPW,), jnp.float32),
            pltpu.VMEM((TPW,), jnp.float32),
            pltpu.VMEM((TPW,), jnp.int32),
            pltpu.VMEM((TPW,), jnp.int32),
            pltpu.VMEM((TPW,), jnp.int32),
            pltpu.VMEM((TPW,), jnp.int32),
            pltpu.VMEM((L,), jnp.int32),
            pltpu.VMEM((NS, L), jnp.int32),
            pltpu.VMEM((HBLK,), jnp.int32),
            pltpu.VMEM((HBLK,), jnp.int32),
            pltpu.SemaphoreType.DMA,
        ],
    )
    return f(sfc_flat, sco_flat, x)


def _c_body(meta_ref, x_ref, w1_ref, w3_ref, w2_ref, o_ref):
    i = pl.program_id(0)

    @pl.when(meta_ref[NBLK + i] > 0)
    def _():
        dn = (((1,), (1,)), ((), ()))
        x = x_ref[...]
        h1 = lax.dot_general(x, w1_ref[0], dn, preferred_element_type=jnp.float32)
        h3 = lax.dot_general(x, w3_ref[0], dn, preferred_element_type=jnp.float32)
        h = h1 * jax.nn.sigmoid(h1) * h3
        o_ref[...] = lax.dot_general(h, w2_ref[0], dn,
                                     preferred_element_type=jnp.float32)


def _run_c(meta, xs, w1, w3, w2):
    grid_spec = pltpu.PrefetchScalarGridSpec(
        num_scalar_prefetch=1,
        grid=(NBLK,),
        in_specs=[
            pl.BlockSpec((BLK, H), lambda i, m: (i, 0)),
            pl.BlockSpec((1, I, H), lambda i, m: (m[i], 0, 0)),
            pl.BlockSpec((1, I, H), lambda i, m: (m[i], 0, 0)),
            pl.BlockSpec((1, H, I), lambda i, m: (m[i], 0, 0)),
        ],
        out_specs=pl.BlockSpec((BLK, H), lambda i, m: (i, 0)),
    )
    return pl.pallas_call(
        _c_body,
        grid_spec=grid_spec,
        out_shape=jax.ShapeDtypeStruct((NROWS, H), jnp.float32),
    )(meta, xs, w1, w3, w2)


CH = 32


def _d_body(os_hbm, rowt_hbm, wts_hbm, sh_hbm, out_hbm,
            i1a, i2a, w1a, w2a, r1_v, r2_v, sh_v, sem):
    wid = lax.axis_index("s") * NC + lax.axis_index("c")
    base = wid * TPW

    def chunk(c, carry):
        row0 = base + c * CH
        pltpu.sync_copy(rowt_hbm.at[0, pl.ds(row0, CH)], i1a)
        pltpu.sync_copy(rowt_hbm.at[1, pl.ds(row0, CH)], i2a)
        cp1 = pltpu.async_copy(os_hbm.at[i1a], r1_v, sem)
        cp2 = pltpu.async_copy(os_hbm.at[i2a], r2_v, sem)
        pltpu.sync_copy(wts_hbm.at[0, pl.ds(row0, CH)], w1a)
        pltpu.sync_copy(wts_hbm.at[1, pl.ds(row0, CH)], w2a)
        pltpu.sync_copy(sh_hbm.at[pl.ds(row0, CH)], sh_v)
        cp1.wait()
        cp2.wait()
        gdn = lax.GatherDimensionNumbers(offset_dims=(),
                                         collapsed_slice_dims=(0,),
                                         start_index_map=(0,))
        wv1 = [w1a[pl.ds(0, L)], w1a[pl.ds(L, L)]]
        wv2 = [w2a[pl.ds(0, L)], w2a[pl.ds(L, L)]]
        for r in range(CH):
            ridx = jnp.full((L, 1), r % L, jnp.int32)
            w1b = lax.gather(wv1[r // L], ridx, gdn, (1,),
                             mode=lax.GatherScatterMode.PROMISE_IN_BOUNDS)
            w2b = lax.gather(wv2[r // L], ridx, gdn, (1,),
                             mode=lax.GatherScatterMode.PROMISE_IN_BOUNDS)

            def colgrp(j, _):
                for k in range(16):
                    sl = pl.ds(j * 256 + k * L, L)
                    r1_v[r, sl] = (w1b * r1_v[r, sl] + w2b * r2_v[r, sl]
                                   + sh_v[r, sl])
                return 0
            lax.fori_loop(0, H // 256, colgrp, 0)
        pltpu.sync_copy(r1_v, out_hbm.at[pl.ds(row0, CH)])
        return 0

    lax.fori_loop(0, TPW // CH, chunk, 0)


def _run_d(outs, rowt, wts, shared):
    mesh = plsc.VectorSubcoreMesh(core_axis_name="c", subcore_axis_name="s")
    f = pl.kernel(
        _d_body,
        out_type=jax.ShapeDtypeStruct((T, H), jnp.float32),
        mesh=mesh,
        compiler_params=pltpu.CompilerParams(needs_layout_passes=False),
        scratch_types=[
            pltpu.VMEM((CH,), jnp.int32),
            pltpu.VMEM((CH,), jnp.int32),
            pltpu.VMEM((CH,), jnp.float32),
            pltpu.VMEM((CH,), jnp.float32),
            pltpu.VMEM((CH, H), jnp.float32),
            pltpu.VMEM((CH, H), jnp.float32),
            pltpu.VMEM((CH, H), jnp.float32),
            pltpu.SemaphoreType.DMA,
        ],
    )
    return f(outs, rowt, wts, shared)


@jax.jit
def kernel(hidden_states, Wg, bias, W1, W3, W2, Ws1, Ws3, Ws2):
    sco, sfc, shared = _run_a(hidden_states, Wg, bias.reshape(1, E),
                              Ws1, Ws3, Ws2)
    xs, wts, rowt, meta, _ = _run_b(sfc.reshape(-1), sco.reshape(-1),
                                    hidden_states)
    outs = _run_c(meta, xs, W1, W3, W2)
    return _run_d(outs, rowt, wts, shared)


# speedup vs baseline: 1.5575x; 1.0030x over previous
import functools

import jax
import jax.numpy as jnp
from jax import lax
from jax.experimental import pallas as pl
from jax.experimental.pallas import tpu as pltpu
from jax.experimental.pallas import tpu_sc as plsc

T = 2048
H = 1024
E = 16
I = 256
TOPK = 2
NG = 4
TG = 2
IS = 256
RSF = 2.5

NC = 2
NS = 16
NW = NC * NS
L = 16
TPW = T // NW
GPW = TPW // L
BLK = 128
HBLK = T * TOPK // (NC * BLK) + E
NBLK = NC * HBLK
HR = HBLK * BLK
NROWS = NBLK * BLK
NEG = -1e30
TA = 256


def _a_body(x_ref, wg_ref, b_ref, ws1_ref, ws3_ref, ws2_ref,
            sco_ref, sfc_ref, shared_ref):
    dn = (((1,), (1,)), ((), ()))
    x = x_ref[...]
    h1 = lax.dot_general(x, ws1_ref[...], dn, preferred_element_type=jnp.float32)
    h3 = lax.dot_general(x, ws3_ref[...], dn, preferred_element_type=jnp.float32)
    h = h1 * jax.nn.sigmoid(h1) * h3
    shared_ref[...] = lax.dot_general(h, ws2_ref[...], dn,
                                      preferred_element_type=jnp.float32)
    logits = lax.dot_general(x, wg_ref[...], dn, preferred_element_type=jnp.float32)
    s = jax.nn.sigmoid(logits)
    sco_ref[...] = s
    sfc_ref[...] = s + b_ref[...]


def _run_a(x, wg, bias2d, ws1, ws3, ws2):
    return pl.pallas_call(
        _a_body,
        grid=(T // TA,),
        in_specs=[
            pl.BlockSpec((TA, H), lambda i: (i, 0)),
            pl.BlockSpec((E, H), lambda i: (0, 0)),
            pl.BlockSpec((1, E), lambda i: (0, 0)),
            pl.BlockSpec((IS, H), lambda i: (0, 0)),
            pl.BlockSpec((IS, H), lambda i: (0, 0)),
            pl.BlockSpec((H, IS), lambda i: (0, 0)),
        ],
        out_specs=[
            pl.BlockSpec((TA, E), lambda i: (i, 0)),
            pl.BlockSpec((TA, E), lambda i: (i, 0)),
            pl.BlockSpec((TA, H), lambda i: (i, 0)),
        ],
        out_shape=[
            jax.ShapeDtypeStruct((T, E), jnp.float32),
            jax.ShapeDtypeStruct((T, E), jnp.float32),
            jax.ShapeDtypeStruct((T, H), jnp.float32),
        ],
    )(x, wg, bias2d, ws1, ws3, ws2)


def _b_body(sfc_hbm, sco_hbm, x_hbm,
            xs_hbm, wts_hbm, rowt_hbm, meta_hbm, cntx_hbm,
            sfc_v, sco_v, xrows_v, idx1_v, idx2_v, wa_v, wb_v,
            p1_v, p2_v, e1_v, e2_v, cnt_v, all_v,
            mbe_v, mac_v, sem):
    core = lax.axis_index("c")
    s = lax.axis_index("s")
    wid = core * NS + s
    base = wid * TPW
    cpx = pltpu.async_copy(x_hbm.at[pl.ds(base, TPW)], xrows_v, sem)
    pltpu.sync_copy(sfc_hbm.at[pl.ds(base * L, TPW * L)], sfc_v)
    pltpu.sync_copy(sco_hbm.at[pl.ds(base * L, TPW * L)], sco_v)

    iota = lax.iota(jnp.int32, L)
    zero = jnp.zeros((L,), jnp.int32)
    cnts = [jnp.int32(0)] * E

    for g in range(GPW):
        col0 = (g * L) * L
        S = []
        SR = []
        for e in range(E):
            idx = iota * L + (col0 + e)
            S.append(plsc.load_gather(sfc_v, [idx]))
            SR.append(plsc.load_gather(sco_v, [idx]))
        G = []
        for gg in range(NG):
            v = S[4 * gg:4 * gg + 4]
            m = None
            for a in range(4):
                for b in range(a + 1, 4):
                    s_ = v[a] + v[b]
                    m = s_ if m is None else jnp.maximum(m, s_)
            G.append(m)
        gsel = []
        for gg in range(NG):
            rank = None
            for gp in range(NG):
                if gp == gg:
                    continue
                c = (G[gp] >= G[gg]) if gp < gg else (G[gp] > G[gg])
                ci = c.astype(jnp.int32)
                rank = ci if rank is None else rank + ci
            gsel.append(rank < TG)
        M = [jnp.where(gsel[e // 4], S[e], NEG) for e in range(E)]
        M1 = M[0]
        for m_ in M[1:]:
            M1 = jnp.maximum(M1, m_)
        i1 = jnp.full((L,), E, jnp.int32)
        for e in range(E - 1, -1, -1):
            i1 = jnp.where(M[e] == M1, e, i1)
        w1 = jnp.zeros((L,), jnp.float32)
        for e in range(E):
            w1 = jnp.where(i1 == e, SR[e], w1)
        M2l = [jnp.where(i1 == e, NEG, M[e]) for e in range(E)]
        M2 = M2l[0]
        for m_ in M2l[1:]:
            M2 = jnp.maximum(M2, m_)
        i2 = jnp.full((L,), E, jnp.int32)
        for e in range(E - 1, -1, -1):
            i2 = jnp.where(M2l[e] == M2, e, i2)
        w2 = jnp.zeros((L,), jnp.float32)
        for e in range(E):
            w2 = jnp.where(i2 == e, SR[e], w2)
        ws = w1 + w2 + 1e-20
        w1n = w1 * RSF / ws
        w2n = w2 * RSF / ws
        wa_v[pl.ds(g * L, L)] = w1n
        wb_v[pl.ds(g * L, L)] = w2n
        pos1 = jnp.zeros((L,), jnp.int32)
        pos2 = jnp.zeros((L,), jnp.int32)
        for e in range(E):
            m1 = i1 == e
            m2 = i2 == e
            occ = (m1 | m2).astype(jnp.int32)
            cums = plsc.cumsum(occ)
            pe = cnts[e] + cums - 1
            pos1 = jnp.where(m1, pe, pos1)
            pos2 = jnp.where(m2, pe, pos2)
            cnts[e] = cnts[e] + jnp.sum(occ)
        p1_v[pl.ds(g * L, L)] = pos1
        p2_v[pl.ds(g * L, L)] = pos2
        e1_v[pl.ds(g * L, L)] = i1
        e2_v[pl.ds(g * L, L)] = i2

    cv = jnp.zeros((L,), jnp.int32)
    for e in range(E):
        cv = jnp.where(iota == e, cnts[e], cv)
    cnt_v[...] = cv
    pltpu.sync_copy(cnt_v, cntx_hbm.at[wid])
    plsc.subcore_barrier()
    pltpu.sync_copy(cntx_hbm.at[pl.ds(core * NS, NS)], all_v)
    tb = jnp.zeros((L,), jnp.int32)
    tot = jnp.zeros((L,), jnp.int32)
    sv = jnp.full((L,), s, jnp.int32)
    for w in range(NS):
        rw = all_v[w, :]
        tot = tot + rw
        tb = tb + jnp.where(sv > w, rw, zero)
    szb = ((tot + (BLK - 1)) // BLK) * BLK
    co = plsc.cumsum(szb)
    basev = co - szb + tb + core * HR

    gdn = lax.GatherDimensionNumbers(offset_dims=(), collapsed_slice_dims=(0,),
                                     start_index_map=(0,))

    def vgather(vec, idx):
        return lax.gather(vec, idx.reshape(L, 1), gdn, (1,),
                          mode=lax.GatherScatterMode.PROMISE_IN_BOUNDS)

    for g in range(GPW):
        i1 = e1_v[pl.ds(g * L, L)]
        i2 = e2_v[pl.ds(g * L, L)]
        r1 = p1_v[pl.ds(g * L, L)] + vgather(basev, i1)
        r2 = p2_v[pl.ds(g * L, L)] + vgather(basev, i2)
        idx1_v[pl.ds(g * L, L)] = jnp.clip(r1, 0, NROWS - 1)
        idx2_v[pl.ds(g * L, L)] = jnp.clip(r2, 0, NROWS - 1)

    cpx.wait()
    cp1 = pltpu.async_copy(xrows_v, xs_hbm.at[idx1_v], sem)
    cp2 = pltpu.async_copy(xrows_v, xs_hbm.at[idx2_v], sem)
    pltpu.sync_copy(idx1_v, rowt_hbm.at[0, pl.ds(base, TPW)])
    pltpu.sync_copy(idx2_v, rowt_hbm.at[1, pl.ds(base, TPW)])
    pltpu.sync_copy(wa_v, wts_hbm.at[0, pl.ds(base, TPW)])
    pltpu.sync_copy(wb_v, wts_hbm.at[1, pl.ds(base, TPW)])
    cp1.wait()
    cp2.wait()

    omax = jnp.max(co)
    for vb in range(HBLK // L):
        brow = (vb * L + iota) * BLK
        be = jnp.zeros((L,), jnp.int32)
        for e in range(E):
            oe = vgather(co, jnp.full((L,), e, jnp.int32))
            be = be + (oe <= brow).astype(jnp.int32)
        be = jnp.minimum(be, E - 1)
        mbe_v[pl.ds(vb * L, L)] = be
        mac_v[pl.ds(vb * L, L)] = (brow < omax).astype(jnp.int32)

    @pl.when(s == 0)
    def _():
        pltpu.sync_copy(mbe_v, meta_hbm.at[pl.ds(core * HBLK, HBLK)])
        pltpu.sync_copy(mac_v, meta_hbm.at[pl.ds(NBLK + core * HBLK, HBLK)])


def _run_b(sfc_flat, sco_flat, x):
    mesh = plsc.VectorSubcoreMesh(core_axis_name="c", subcore_axis_name="s")
    f = pl.kernel(
        _b_body,
        out_type=[
            jax.ShapeDtypeStruct((NROWS, H), jnp.float32),
            jax.ShapeDtypeStruct((2, T), jnp.float32),
            jax.ShapeDtypeStruct((2, T), jnp.int32),
            jax.ShapeDtypeStruct((2 * NBLK,), jnp.int32),
            jax.ShapeDtypeStruct((NW, L), jnp.int32),
        ],
        mesh=mesh,
        compiler_params=pltpu.CompilerParams(needs_layout_passes=False),
        scratch_types=[
            pltpu.VMEM((TPW * L,), jnp.float32),
            pltpu.VMEM((TPW * L,), jnp.float32),
            pltpu.VMEM((TPW, H), jnp.float32),
            pltpu.VMEM((TPW,), jnp.int32),
            pltpu.VMEM((TPW,), jnp.int32),
            pltpu.VMEM((TPW,), jnp.float32),
            pltpu.VMEM((TPW,), jnp.float32),
            pltpu.VMEM((TPW,), jnp.int32),
            pltpu.VMEM((TPW,), jnp.int32),
            pltpu.VMEM((TPW,), jnp.int32),
            pltpu.VMEM((TPW,), jnp.int32),
            pltpu.VMEM((L,), jnp.int32),
            pltpu.VMEM((NS, L), jnp.int32),
            pltpu.VMEM((HBLK,), jnp.int32),
            pltpu.VMEM((HBLK,), jnp.int32),
            pltpu.SemaphoreType.DMA,
        ],
    )
    return f(sfc_flat, sco_flat, x)


def _c_body(meta_ref, x_ref, w1_ref, w3_ref, w2_ref, o_ref):
    i = pl.program_id(0)

    @pl.when(meta_ref[NBLK + i] > 0)
    def _():
        dn = (((1,), (1,)), ((), ()))
        x = x_ref[...]
        h1 = lax.dot_general(x, w1_ref[0], dn, preferred_element_type=jnp.float32)
        h3 = lax.dot_general(x, w3_ref[0], dn, preferred_element_type=jnp.float32)
        h = h1 * jax.nn.sigmoid(h1) * h3
        o_ref[...] = lax.dot_general(h, w2_ref[0], dn,
                                     preferred_element_type=jnp.float32)


def _run_c(meta, xs, w1, w3, w2):
    grid_spec = pltpu.PrefetchScalarGridSpec(
        num_scalar_prefetch=1,
        grid=(NBLK,),
        in_specs=[
            pl.BlockSpec((BLK, H), lambda i, m: (i, 0)),
            pl.BlockSpec((1, I, H), lambda i, m: (m[i], 0, 0)),
            pl.BlockSpec((1, I, H), lambda i, m: (m[i], 0, 0)),
            pl.BlockSpec((1, H, I), lambda i, m: (m[i], 0, 0)),
        ],
        out_specs=pl.BlockSpec((BLK, H), lambda i, m: (i, 0)),
    )
    return pl.pallas_call(
        _c_body,
        grid_spec=grid_spec,
        out_shape=jax.ShapeDtypeStruct((NROWS, H), jnp.float32),
    )(meta, xs, w1, w3, w2)


CH = 32


def _d_body(os_hbm, rowt_hbm, wts_hbm, sh_hbm, out_hbm,
            i1a, i2a, w1a, w2a, r1_v, r2_v, sh_v, sem):
    wid = lax.axis_index("s") * NC + lax.axis_index("c")
    base = wid * TPW

    def chunk(c, carry):
        row0 = base + c * CH
        pltpu.sync_copy(rowt_hbm.at[0, pl.ds(row0, CH)], i1a)
        pltpu.sync_copy(rowt_hbm.at[1, pl.ds(row0, CH)], i2a)
        cp1 = pltpu.async_copy(os_hbm.at[i1a], r1_v, sem)
        cp2 = pltpu.async_copy(os_hbm.at[i2a], r2_v, sem)
        pltpu.sync_copy(wts_hbm.at[0, pl.ds(row0, CH)], w1a)
        pltpu.sync_copy(wts_hbm.at[1, pl.ds(row0, CH)], w2a)
        pltpu.sync_copy(sh_hbm.at[pl.ds(row0, CH)], sh_v)
        cp1.wait()
        cp2.wait()
        gdn = lax.GatherDimensionNumbers(offset_dims=(),
                                         collapsed_slice_dims=(0,),
                                         start_index_map=(0,))
        wv1 = [w1a[pl.ds(0, L)], w1a[pl.ds(L, L)]]
        wv2 = [w2a[pl.ds(0, L)], w2a[pl.ds(L, L)]]
        for r in range(CH):
            ridx = jnp.full((L, 1), r % L, jnp.int32)
            w1b = lax.gather(wv1[r // L], ridx, gdn, (1,),
                             mode=lax.GatherScatterMode.PROMISE_IN_BOUNDS)
            w2b = lax.gather(wv2[r // L], ridx, gdn, (1,),
                             mode=lax.GatherScatterMode.PROMISE_IN_BOUNDS)

            def colgrp(j, _):
                for k in range(16):
                    sl = pl.ds(j * 256 + k * L, L)
                    r1_v[r, sl] = (w1b * r1_v[r, sl] + w2b * r2_v[r, sl]
                                   + sh_v[r, sl])
                return 0
            lax.fori_loop(0, H // 256, colgrp, 0)
        pltpu.sync_copy(r1_v, out_hbm.at[pl.ds(row0, CH)])
        return 0

    lax.fori_loop(0, TPW // CH, chunk, 0)


def _run_d(outs, rowt, wts, shared):
    mesh = plsc.VectorSubcoreMesh(core_axis_name="c", subcore_axis_name="s")
    f = pl.kernel(
        _d_body,
        out_type=jax.ShapeDtypeStruct((T, H), jnp.float32),
        mesh=mesh,
        compiler_params=pltpu.CompilerParams(needs_layout_passes=False),
        scratch_types=[
            pltpu.VMEM((CH,), jnp.int32),
            pltpu.VMEM((CH,), jnp.int32),
            pltpu.VMEM((CH,), jnp.float32),
            pltpu.VMEM((CH,), jnp.float32),
            pltpu.VMEM((CH, H), jnp.float32),
            pltpu.VMEM((CH, H), jnp.float32),
            pltpu.VMEM((CH, H), jnp.float32),
            pltpu.SemaphoreType.DMA,
        ],
    )
    return f(outs, rowt, wts, shared)


@jax.jit
def kernel(hidden_states, Wg, bias, W1, W3, W2, Ws1, Ws3, Ws2):
    sco, sfc, shared = _run_a(hidden_states, Wg, bias.reshape(1, E),
                              Ws1, Ws3, Ws2)
    xs, wts, rowt, meta, _ = _run_b(sfc.reshape(-1), sco.reshape(-1),
                                    hidden_states)
    outs = _run_c(meta, xs, W1, W3, W2)
    return _run_d(outs, rowt, wts, shared)


# speedup vs baseline: 1.5600x; 1.0016x over previous
import jax
import jax.numpy as jnp
from jax import lax
from jax.experimental import pallas as pl
from jax.experimental.pallas import tpu as pltpu
from jax.experimental.pallas import tpu_sc as plsc

T = 2048
H = 1024
E = 16
I = 256
TOPK = 2
NG = 4
TG = 2
IS = 256
RSF = 2.5

NC = 2
NS = 16
NW = NC * NS
L = 16
TPW = T // NW
GPW = TPW // L
BLK = 128
HBLK = T * TOPK // (NC * BLK) + E
NBLK = NC * HBLK
HR = HBLK * BLK
NROWS = NBLK * BLK
NEG = -1e30
TA = 256


def _a_body(x_ref, wg_ref, b_ref, ws1_ref, ws3_ref, ws2_ref,
            sco_ref, sfc_ref, shared_ref):
    dn = (((1,), (1,)), ((), ()))
    x = x_ref[...]
    h1 = lax.dot_general(x, ws1_ref[...], dn, preferred_element_type=jnp.float32)
    h3 = lax.dot_general(x, ws3_ref[...], dn, preferred_element_type=jnp.float32)
    h = h1 * jax.nn.sigmoid(h1) * h3
    shared_ref[...] = lax.dot_general(h, ws2_ref[...], dn,
                                      preferred_element_type=jnp.float32)
    logits = lax.dot_general(x, wg_ref[...], dn, preferred_element_type=jnp.float32)
    s = jax.nn.sigmoid(logits)
    sco_ref[...] = s
    sfc_ref[...] = s + b_ref[...]


def _run_a(x, wg, bias2d, ws1, ws3, ws2):
    return pl.pallas_call(
        _a_body,
        grid=(T // TA,),
        in_specs=[
            pl.BlockSpec((TA, H), lambda i: (i, 0)),
            pl.BlockSpec((E, H), lambda i: (0, 0)),
            pl.BlockSpec((1, E), lambda i: (0, 0)),
            pl.BlockSpec((IS, H), lambda i: (0, 0)),
            pl.BlockSpec((IS, H), lambda i: (0, 0)),
            pl.BlockSpec((H, IS), lambda i: (0, 0)),
        ],
        out_specs=[
            pl.BlockSpec((TA, E), lambda i: (i, 0)),
            pl.BlockSpec((TA, E), lambda i: (i, 0)),
            pl.BlockSpec((TA, H), lambda i: (i, 0)),
        ],
        out_shape=[
            jax.ShapeDtypeStruct((T, E), jnp.float32),
            jax.ShapeDtypeStruct((T, E), jnp.float32),
            jax.ShapeDtypeStruct((T, H), jnp.float32),
        ],
    )(x, wg, bias2d, ws1, ws3, ws2)


def _b_body(sfc_hbm, sco_hbm, x_hbm,
            xs_hbm, wts_hbm, rowt_hbm, meta_hbm, cntx_hbm,
            sfc_v, sco_v, xrows_v, idx1_v, idx2_v, wa_v, wb_v,
            p1_v, p2_v, e1_v, e2_v, cnt_v, all_v,
            mbe_v, mac_v, sem):
    core = lax.axis_index("c")
    s = lax.axis_index("s")
    wid = core * NS + s
    base = wid * TPW
    cpx = pltpu.async_copy(x_hbm.at[pl.ds(base, TPW)], xrows_v, sem)
    pltpu.sync_copy(sfc_hbm.at[pl.ds(base * L, TPW * L)], sfc_v)
    pltpu.sync_copy(sco_hbm.at[pl.ds(base * L, TPW * L)], sco_v)

    iota = lax.iota(jnp.int32, L)
    zero = jnp.zeros((L,), jnp.int32)
    cnts = [jnp.int32(0)] * E

    for g in range(GPW):
        col0 = (g * L) * L
        S = []
        SR = []
        for e in range(E):
            idx = iota * L + (col0 + e)
            S.append(plsc.load_gather(sfc_v, [idx]))
            SR.append(plsc.load_gather(sco_v, [idx]))
        G = []
        for gg in range(NG):
            v = S[4 * gg:4 * gg + 4]
            m = None
            for a in range(4):
                for b in range(a + 1, 4):
                    s_ = v[a] + v[b]
                    m = s_ if m is None else jnp.maximum(m, s_)
            G.append(m)
        gsel = []
        for gg in range(NG):
            rank = None
            for gp in range(NG):
                if gp == gg:
                    continue
                c = (G[gp] >= G[gg]) if gp < gg else (G[gp] > G[gg])
                ci = c.astype(jnp.int32)
                rank = ci if rank is None else rank + ci
            gsel.append(rank < TG)
        M = [jnp.where(gsel[e // 4], S[e], NEG) for e in range(E)]
        M1 = M[0]
        for m_ in M[1:]:
            M1 = jnp.maximum(M1, m_)
        i1 = jnp.full((L,), E, jnp.int32)
        for e in range(E - 1, -1, -1):
            i1 = jnp.where(M[e] == M1, e, i1)
        w1 = jnp.zeros((L,), jnp.float32)
        for e in range(E):
            w1 = jnp.where(i1 == e, SR[e], w1)
        M2l = [jnp.where(i1 == e, NEG, M[e]) for e in range(E)]
        M2 = M2l[0]
        for m_ in M2l[1:]:
            M2 = jnp.maximum(M2, m_)
        i2 = jnp.full((L,), E, jnp.int32)
        for e in range(E - 1, -1, -1):
            i2 = jnp.where(M2l[e] == M2, e, i2)
        w2 = jnp.zeros((L,), jnp.float32)
        for e in range(E):
            w2 = jnp.where(i2 == e, SR[e], w2)
        ws = w1 + w2 + 1e-20
        w1n = w1 * RSF / ws
        w2n = w2 * RSF / ws
        wa_v[pl.ds(g * L, L)] = w1n
        wb_v[pl.ds(g * L, L)] = w2n
        pos1 = jnp.zeros((L,), jnp.int32)
        pos2 = jnp.zeros((L,), jnp.int32)
        for e in range(E):
            m1 = i1 == e
            m2 = i2 == e
            occ = (m1 | m2).astype(jnp.int32)
            cums = plsc.cumsum(occ)
            pe = cnts[e] + cums - 1
            pos1 = jnp.where(m1, pe, pos1)
            pos2 = jnp.where(m2, pe, pos2)
            cnts[e] = cnts[e] + jnp.sum(occ)
        p1_v[pl.ds(g * L, L)] = pos1
        p2_v[pl.ds(g * L, L)] = pos2
        e1_v[pl.ds(g * L, L)] = i1
        e2_v[pl.ds(g * L, L)] = i2

    cv = jnp.zeros((L,), jnp.int32)
    for e in range(E):
        cv = jnp.where(iota == e, cnts[e], cv)
    cnt_v[...] = cv
    pltpu.sync_copy(cnt_v, cntx_hbm.at[wid])
    plsc.subcore_barrier()
    pltpu.sync_copy(cntx_hbm.at[pl.ds(core * NS, NS)], all_v)
    tb = jnp.zeros((L,), jnp.int32)
    tot = jnp.zeros((L,), jnp.int32)
    sv = jnp.full((L,), s, jnp.int32)
    for w in range(NS):
        rw = all_v[w, :]
        tot = tot + rw
        tb = tb + jnp.where(sv > w, rw, zero)
    szb = ((tot + (BLK - 1)) // BLK) * BLK
    co = plsc.cumsum(szb)
    basev = co - szb + tb + core * HR

    gdn = lax.GatherDimensionNumbers(offset_dims=(), collapsed_slice_dims=(0,),
                                     start_index_map=(0,))

    def vgather(vec, idx):
        return lax.gather(vec, idx.reshape(L, 1), gdn, (1,),
                          mode=lax.GatherScatterMode.PROMISE_IN_BOUNDS)

    for g in range(GPW):
        i1 = e1_v[pl.ds(g * L, L)]
        i2 = e2_v[pl.ds(g * L, L)]
        r1 = p1_v[pl.ds(g * L, L)] + vgather(basev, i1)
        r2 = p2_v[pl.ds(g * L, L)] + vgather(basev, i2)
        idx1_v[pl.ds(g * L, L)] = jnp.clip(r1, 0, NROWS - 1)
        idx2_v[pl.ds(g * L, L)] = jnp.clip(r2, 0, NROWS - 1)

    cpx.wait()
    cp1 = pltpu.async_copy(xrows_v, xs_hbm.at[idx1_v], sem)
    cp2 = pltpu.async_copy(xrows_v, xs_hbm.at[idx2_v], sem)
    pltpu.sync_copy(idx1_v, rowt_hbm.at[0, pl.ds(base, TPW)])
    pltpu.sync_copy(idx2_v, rowt_hbm.at[1, pl.ds(base, TPW)])
    pltpu.sync_copy(wa_v, wts_hbm.at[0, pl.ds(base, TPW)])
    pltpu.sync_copy(wb_v, wts_hbm.at[1, pl.ds(base, TPW)])
    cp1.wait()
    cp2.wait()

    omax = jnp.max(co)
    for vb in range(HBLK // L):
        brow = (vb * L + iota) * BLK
        be = jnp.zeros((L,), jnp.int32)
        for e in range(E):
            oe = vgather(co, jnp.full((L,), e, jnp.int32))
            be = be + (oe <= brow).astype(jnp.int32)
        be = jnp.minimum(be, E - 1)
        mbe_v[pl.ds(vb * L, L)] = be
        mac_v[pl.ds(vb * L, L)] = (brow < omax).astype(jnp.int32)

    @pl.when(s == 0)
    def _():
        pltpu.sync_copy(mbe_v, meta_hbm.at[pl.ds(core * HBLK, HBLK)])
        pltpu.sync_copy(mac_v, meta_hbm.at[pl.ds(NBLK + core * HBLK, HBLK)])


def _run_b(sfc_flat, sco_flat, x):
    mesh = plsc.VectorSubcoreMesh(core_axis_name="c", subcore_axis_name="s")
    f = pl.kernel(
        _b_body,
        out_type=[
            jax.ShapeDtypeStruct((NROWS, H), jnp.float32),
            jax.ShapeDtypeStruct((2, T), jnp.float32),
            jax.ShapeDtypeStruct((2, T), jnp.int32),
            jax.ShapeDtypeStruct((2 * NBLK,), jnp.int32),
            jax.ShapeDtypeStruct((NW, L), jnp.int32),
        ],
        mesh=mesh,
        compiler_params=pltpu.CompilerParams(needs_layout_passes=False),
        scratch_types=[
            pltpu.VMEM((TPW * L,), jnp.float32),
            pltpu.VMEM((TPW * L,), jnp.float32),
            pltpu.VMEM((TPW, H), jnp.float32),
            pltpu.VMEM((TPW,), jnp.int32),
            pltpu.VMEM((TPW,), jnp.int32),
            pltpu.VMEM((TPW,), jnp.float32),
            pltpu.VMEM((TPW,), jnp.float32),
            pltpu.VMEM((TPW,), jnp.int32),
            pltpu.VMEM((TPW,), jnp.int32),
            pltpu.VMEM((TPW,), jnp.int32),
            pltpu.VMEM((TPW,), jnp.int32),
            pltpu.VMEM((L,), jnp.int32),
            pltpu.VMEM((NS, L), jnp.int32),
            pltpu.VMEM((HBLK,), jnp.int32),
            pltpu.VMEM((HBLK,), jnp.int32),
            pltpu.SemaphoreType.DMA,
        ],
    )
    return f(sfc_flat, sco_flat, x)


def _c_body(meta_ref, x_ref, w1_ref, w3_ref, w2_ref, o_ref):
    i = pl.program_id(0)

    @pl.when(meta_ref[NBLK + i] > 0)
    def _():
        dn = (((1,), (1,)), ((), ()))
        x = x_ref[...]
        h1 = lax.dot_general(x, w1_ref[0], dn, preferred_element_type=jnp.float32)
        h3 = lax.dot_general(x, w3_ref[0], dn, preferred_element_type=jnp.float32)
        h = h1 * jax.nn.sigmoid(h1) * h3
        o_ref[...] = lax.dot_general(h, w2_ref[0], dn,
                                     preferred_element_type=jnp.float32)


def _run_c(meta, xs, w1, w3, w2):
    grid_spec = pltpu.PrefetchScalarGridSpec(
        num_scalar_prefetch=1,
        grid=(NBLK,),
        in_specs=[
            pl.BlockSpec((BLK, H), lambda i, m: (i, 0)),
            pl.BlockSpec((1, I, H), lambda i, m: (m[i], 0, 0)),
            pl.BlockSpec((1, I, H), lambda i, m: (m[i], 0, 0)),
            pl.BlockSpec((1, H, I), lambda i, m: (m[i], 0, 0)),
        ],
        out_specs=pl.BlockSpec((BLK, H), lambda i, m: (i, 0)),
    )
    return pl.pallas_call(
        _c_body,
        grid_spec=grid_spec,
        out_shape=jax.ShapeDtypeStruct((NROWS, H), jnp.float32),
    )(meta, xs, w1, w3, w2)


CH = 32


def _d_body(os_hbm, rowt_hbm, wts_hbm, sh_hbm, out_hbm,
            i1a, i2a, w1a, w2a, r1_v, r2_v, sh_v, sem):
    wid = lax.axis_index("s") * NC + lax.axis_index("c")
    base = wid * TPW

    def chunk(c, carry):
        row0 = base + c * CH
        pltpu.sync_copy(rowt_hbm.at[0, pl.ds(row0, CH)], i1a)
        pltpu.sync_copy(rowt_hbm.at[1, pl.ds(row0, CH)], i2a)
        cp1 = pltpu.async_copy(os_hbm.at[i1a], r1_v, sem)
        cp2 = pltpu.async_copy(os_hbm.at[i2a], r2_v, sem)
        pltpu.sync_copy(wts_hbm.at[0, pl.ds(row0, CH)], w1a)
        pltpu.sync_copy(wts_hbm.at[1, pl.ds(row0, CH)], w2a)
        pltpu.sync_copy(sh_hbm.at[pl.ds(row0, CH)], sh_v)
        cp1.wait()
        cp2.wait()
        gdn = lax.GatherDimensionNumbers(offset_dims=(),
                                         collapsed_slice_dims=(0,),
                                         start_index_map=(0,))
        wv1 = [w1a[pl.ds(0, L)], w1a[pl.ds(L, L)]]
        wv2 = [w2a[pl.ds(0, L)], w2a[pl.ds(L, L)]]
        for r in range(CH):
            ridx = jnp.full((L, 1), r % L, jnp.int32)
            w1b = lax.gather(wv1[r // L], ridx, gdn, (1,),
                             mode=lax.GatherScatterMode.PROMISE_IN_BOUNDS)
            w2b = lax.gather(wv2[r // L], ridx, gdn, (1,),
                             mode=lax.GatherScatterMode.PROMISE_IN_BOUNDS)

            def colgrp(j, _):
                for k in range(16):
                    sl = pl.ds(j * 256 + k * L, L)
                    r1_v[r, sl] = (w1b * r1_v[r, sl] + w2b * r2_v[r, sl]
                                   + sh_v[r, sl])
                return 0
            lax.fori_loop(0, H // 256, colgrp, 0)
        pltpu.sync_copy(r1_v, out_hbm.at[pl.ds(row0, CH)])
        return 0

    lax.fori_loop(0, TPW // CH, chunk, 0)


def _run_d(outs, rowt, wts, shared):
    mesh = plsc.VectorSubcoreMesh(core_axis_name="c", subcore_axis_name="s")
    f = pl.kernel(
        _d_body,
        out_type=jax.ShapeDtypeStruct((T, H), jnp.float32),
        mesh=mesh,
        compiler_params=pltpu.CompilerParams(needs_layout_passes=False),
        scratch_types=[
            pltpu.VMEM((CH,), jnp.int32),
            pltpu.VMEM((CH,), jnp.int32),
            pltpu.VMEM((CH,), jnp.float32),
            pltpu.VMEM((CH,), jnp.float32),
            pltpu.VMEM((CH, H), jnp.float32),
            pltpu.VMEM((CH, H), jnp.float32),
            pltpu.VMEM((CH, H), jnp.float32),
            pltpu.SemaphoreType.DMA,
        ],
    )
    return f(outs, rowt, wts, shared)


@jax.jit
def kernel(hidden_states, Wg, bias, W1, W3, W2, Ws1, Ws3, Ws2):
    sco, sfc, shared = _run_a(hidden_states, Wg, bias.reshape(1, E),
                              Ws1, Ws3, Ws2)
    xs, wts, rowt, meta, _ = _run_b(sfc.reshape(-1), sco.reshape(-1),
                                    hidden_states)
    outs = _run_c(meta, xs, W1, W3, W2)
    return _run_d(outs, rowt, wts, shared)
